# baseline (speedup 1.0000x reference)
import jax
import jax.numpy as jnp
from jax import lax
from jax.experimental import pallas as pl
from jax.experimental.pallas import tpu as pltpu

B = 20
P = 1000
K = 15
KS = 5
FN = 10
M = P // 5
NA = 4
G5 = 5
NQ = B // G5

_BIG = 1e9
_bf16 = jnp.bfloat16


_EXCLUDED = 0x7FFF0000


def _pack(cur, ili):
    bits = jax.lax.bitcast_convert_type(cur, jnp.int32)
    return (bits & -1024) | ili


def _argmin_step(curp, ili):
    mv = jnp.min(curp, axis=1, keepdims=True)
    hit = curp == mv
    m32 = jnp.where(hit, 1.0, 0.0)
    curp = jnp.where(hit, jnp.full_like(curp, _EXCLUDED), curp)
    return curp, mv & 1023, m32


def _gather(m32, src):
    return jnp.dot(m32, src)


def _stage_a(g, pcol_ref, prow_ref, wm_ref, w1_ref, b1_ref, w2_ref, b2_ref,
             fdd_s):
    f32 = jnp.float32
    pc = pcol_ref[0]
    pr = prow_ref[:, 0, 0, :]
    ili = lax.broadcasted_iota(jnp.int32, (P, P), 1)
    isi = lax.broadcasted_iota(jnp.int32, (P, P), 0)
    d2 = ((pc[:, 0:1] - pr[0:1, :]) ** 2
          + (pc[:, 1:2] - pr[1:2, :]) ** 2
          + (pc[:, 2:3] - pr[2:3, :]) ** 2)
    cur = d2 + jnp.where(ili == isi, _BIG, 0.0)
    curp = _pack(cur, ili)

    l15 = lax.broadcasted_iota(jnp.int32, (1, 3 * KS), 1)
    gridf = (l15 % KS).astype(f32)
    s3 = lax.broadcasted_iota(jnp.int32, (3, 3 * KS), 0)
    rsel = (l15 // KS == s3).astype(f32)

    facc = jnp.zeros((P, 3 * FN), f32)
    idxs = []
    for _ in range(K):
        curp, idxi, mbf = _argmin_step(curp, ili)
        idxs.append(idxi)
        rel = _gather(mbf, pc) - pc
        u = (jnp.tanh(rel) + 1.0) * (0.5 * (KS - 1))
        urep = jnp.dot(u, rsel)
        basis = jnp.maximum(0.0, 1.0 - jnp.abs(urep - gridf))
        facc = facc + jnp.dot(basis, wm_ref[...])
    f3d = jax.nn.sigmoid(facc * (1.0 / K))

    w1 = w1_ref[...]
    gw = (jnp.dot(f3d, w1[:3 * FN, :])
          + jnp.dot(pc, w1[3 * FN:, :]))
    shift = jnp.dot(pc, w1[3 * FN:, :]) - b1_ref[...]
    w2 = w2_ref[...]
    b2 = b2_ref[...]
    fddm = jnp.full((P, 20), -jnp.inf, f32)
    for k in range(K):
        mbf = jnp.where(ili == idxs[k], 1.0, 0.0)
        h = jnp.maximum(_gather(mbf, gw) - shift, 0.0)
        h2 = jnp.maximum(jnp.dot(h, w2) + b2, 0.0)
        fddm = jnp.maximum(fddm, h2)
    fdd_s[pl.ds(g * P, P), :] = jax.nn.sigmoid(fddm)


def _fps(prow_ref, px2_s, py2_s, pz2_s):
    f32 = jnp.float32
    px = prow_ref[0]
    py = prow_ref[1]
    pz = prow_ref[2]
    lane = lax.broadcasted_iota(jnp.int32, (B, P), 1)
    lanerev = 1023 - lane
    lane2 = lax.broadcasted_iota(jnp.int32, (B, M), 1)
    pst = jnp.concatenate([px, py, pz], axis=0)
    cx0 = px[:, 0:1]
    cy0 = py[:, 0:1]
    cz0 = pz[:, 0:1]
    mind = (px - cx0) ** 2 + (py - cy0) ** 2 + (pz - cz0) ** 2
    p2x = jnp.where(lane2 == 0, cx0, 0.0)
    p2y = jnp.where(lane2 == 0, cy0, 0.0)
    p2z = jnp.where(lane2 == 0, cz0, 0.0)

    def body(i, st):
        mind, p2x, p2y, p2z = st
        mp = ((jax.lax.bitcast_convert_type(mind, jnp.int32)
               & -1024) | lanerev)
        mv = jnp.max(mp, axis=1, keepdims=True)
        ohf = jnp.where(mp == mv, 1.0, 0.0)
        oh3 = jnp.concatenate([ohf, ohf, ohf], axis=0)
        c3 = jnp.sum(pst * oh3, axis=1, keepdims=True)
        cx = c3[:B]
        cy = c3[B:2 * B]
        cz = c3[2 * B:]
        d = (px - cx) ** 2 + (py - cy) ** 2 + (pz - cz) ** 2
        mind = jnp.minimum(mind, d)
        sel = lane2 == i
        p2x = jnp.where(sel, cx, p2x)
        p2y = jnp.where(sel, cy, p2y)
        p2z = jnp.where(sel, cz, p2z)
        return (mind, p2x, p2y, p2z)

    _, p2x, p2y, p2z = lax.fori_loop(1, M, body, (mind, p2x, p2y, p2z),
                                     unroll=4)
    px2_s[...] = p2x
    py2_s[...] = p2y
    pz2_s[...] = p2z


def _stage_c(q, w3_ref, b3_ref, w4_ref, b4_ref, fdd_s, px2_s, py2_s, pz2_s,
             ys_s):
    f32 = jnp.float32
    R = G5 * M
    isr = lax.broadcasted_iota(jnp.int32, (R, M), 0)
    ili = lax.broadcasted_iota(jnp.int32, (R, M), 1)
    eyeb = isr % M == ili
    iob = lax.broadcasted_iota(jnp.int32, (B, 1), 0)
    ior = lax.broadcasted_iota(jnp.int32, (R, 1), 0)

    def rowstack(ref):
        acc = jnp.zeros((R, M), f32)
        for g in range(G5):
            rsel = iob == (q * G5 + g)
            row = jnp.sum(jnp.where(rsel, ref[...], 0.0), axis=0,
                          keepdims=True)
            acc = acc + jnp.where(ior // M == g, row, 0.0)
        return acc

    prx = rowstack(px2_s)
    pry = rowstack(py2_s)
    prz = rowstack(pz2_s)
    pcx = jnp.sum(jnp.where(eyeb, prx, 0.0), axis=1, keepdims=True)
    pcy = jnp.sum(jnp.where(eyeb, pry, 0.0), axis=1, keepdims=True)
    pcz = jnp.sum(jnp.where(eyeb, prz, 0.0), axis=1, keepdims=True)
    d2 = (pcx - prx) ** 2 + (pcy - pry) ** 2 + (pcz - prz) ** 2
    curp = _pack(d2 + jnp.where(eyeb, _BIG, 0.0), ili)

    sub_all = fdd_s[pl.ds(q * R, R), :]
    sub5 = jnp.concatenate(
        [sub_all[M * g:M * (g + 1), :] for g in range(G5)], axis=1)
    macc = jnp.zeros((R, 20), f32)
    mmax = jnp.full((R, 20), -jnp.inf, f32)
    for _ in range(K):
        curp, _, mbf = _argmin_step(curp, ili)
        g100 = _gather(mbf, sub5)
        gg = jnp.concatenate(
            [g100[M * g:M * (g + 1), 20 * g:20 * (g + 1)]
             for g in range(G5)], axis=0)
        macc = macc + gg
        mmax = jnp.maximum(mmax, gg)
    x2 = jnp.concatenate([sub_all, macc * (1.0 / K), mmax], axis=1)
    h = jnp.maximum(jnp.dot(x2, w3_ref[...]) + b3_ref[...], 0.0)
    h2 = jnp.maximum(jnp.dot(h, w4_ref[...]) + b4_ref[...], 0.0)
    f2 = jax.nn.sigmoid(h2)
    mmean = jnp.where(
        lax.broadcasted_iota(jnp.int32, (G5, R), 0)
        == lax.broadcasted_iota(jnp.int32, (G5, R), 1) // M,
        1.0 / M, 0.0)
    ysq = jnp.dot(mmean, f2)
    upd = jnp.zeros((B, 32), f32)
    for g in range(G5):
        upd = upd + jnp.where(iob == (q * G5 + g), ysq[g:g + 1, :], 0.0)
    ys_s[...] = ys_s[...] + upd


def _head(wn1_ref, bn1_ref, wn2_ref, bn2_ref, ys_s, out_ref):
    ys = ys_s[...]
    y1 = jnp.dot(ys, wn1_ref[...]) + bn1_ref[...]
    y1 = jnp.where(y1 > 0, y1, jnp.exp(jnp.minimum(y1, 0.0)) - 1.0)
    z = jnp.dot(y1, wn2_ref[...]) + bn2_ref[...]
    s = z - jnp.max(z, axis=1, keepdims=True)
    out_ref[...] = s - jnp.log(jnp.sum(jnp.exp(s), axis=1, keepdims=True))


def _body(pcol_ref, prow_all_ref, prow_g_ref, wm_ref, w1_ref, b1_ref,
          w2_ref, b2_ref, w3_ref, b3_ref, w4_ref, b4_ref,
          wn1_ref, bn1_ref, wn2_ref, bn2_ref,
          out_ref, fdd_s, px2_s, py2_s, pz2_s, ys_s):
    s = pl.program_id(0)

    @pl.when(s < NA)
    def _():
        _stage_a(s, pcol_ref, prow_g_ref, wm_ref, w1_ref, b1_ref,
                 w2_ref, b2_ref, fdd_s)

    @pl.when(s == NA)
    def _():
        ys_s[...] = jnp.zeros((B, 32), jnp.float32)
        _fps(prow_all_ref, px2_s, py2_s, pz2_s)

    @pl.when(s > NA)
    def _():
        _stage_c(s - (NA + 1), w3_ref, b3_ref, w4_ref, b4_ref,
                 fdd_s, px2_s, py2_s, pz2_s, ys_s)

    @pl.when(s == NA + NQ)
    def _():
        _head(wn1_ref, bn1_ref, wn2_ref, bn2_ref, ys_s, out_ref)


def kernel(pos, edge_index, batch, Wsp, W1, b1, W2, b2, W3, b3, W4, b4,
           Wn1, bn1, Wn2, bn2):
    del edge_index, batch
    f32 = jnp.float32
    pos = pos.astype(f32)
    posg = pos.reshape(B, P, 3)
    pg4 = posg[:NA]
    pall_row = jnp.transpose(posg, (2, 0, 1))
    pg4_row = pall_row[:, :NA, None, :]

    w_cjf = jnp.transpose(Wsp.astype(f32), (2, 1, 0))
    wmat = (w_cjf[:, :, :, None]
            * jnp.eye(3, dtype=f32)[:, None, None, :]).reshape(3 * KS, 3 * FN)

    nsteps = NA + 1 + NQ
    whole = lambda shape: pl.BlockSpec(shape, lambda s: (0,) * len(shape))
    out = pl.pallas_call(
        _body,
        grid=(nsteps,),
        in_specs=[
            pl.BlockSpec((1, P, 3), lambda s: (jnp.minimum(s, NA - 1), 0, 0)),
            whole((3, B, P)),
            pl.BlockSpec((3, 1, 1, P),
                         lambda s: (0, jnp.minimum(s, NA - 1), 0, 0)),
            whole((3 * KS, 3 * FN)),
            whole((3 * FN + 3, 20)),
            whole((1, 20)),
            whole((20, 20)),
            whole((1, 20)),
            whole((60, 64)),
            whole((1, 64)),
            whole((64, 32)),
            whole((1, 32)),
            whole((32, 256)),
            whole((1, 256)),
            whole((256, 40)),
            whole((1, 40)),
        ],
        out_specs=pl.BlockSpec((B, 40), lambda s: (0, 0)),
        out_shape=jax.ShapeDtypeStruct((B, 40), f32),
        scratch_shapes=[
            pltpu.VMEM((NA * P, 20), f32),
            pltpu.VMEM((B, M), f32),
            pltpu.VMEM((B, M), f32),
            pltpu.VMEM((B, M), f32),
            pltpu.VMEM((B, 32), f32),
        ],
    )(pg4, pall_row, pg4_row, wmat, W1, b1.reshape(1, 20), W2,
      b2.reshape(1, 20), W3, b3.reshape(1, 64), W4, b4.reshape(1, 32),
      Wn1, bn1.reshape(1, 256), Wn2, bn2.reshape(1, 40))
    return out


# speedup vs baseline: 1.2745x; 1.2745x over previous
import jax
import jax.numpy as jnp
from jax import lax
from jax.experimental import pallas as pl
from jax.experimental.pallas import tpu as pltpu

B = 20
P = 1000
K = 15
KS = 5
FN = 10
M = P // 5
NA = 4

_BIG = 1e9
_bf16 = jnp.bfloat16


_EXCLUDED = 0x7FFF0000


def _pack(cur, ili):
    bits = jax.lax.bitcast_convert_type(cur, jnp.int32)
    return (bits & -1024) | ili


def _argmin_step(curp, ili):
    mv = jnp.min(curp, axis=1, keepdims=True)
    hit = curp == mv
    m32 = jnp.where(hit, 1.0, 0.0)
    curp = jnp.where(hit, jnp.full_like(curp, _EXCLUDED), curp)
    return curp, mv & 1023, m32


def _gather(m32, src):
    return jnp.dot(m32, src)


def _stage_a(g, pcol_ref, prow_ref, wm_ref, w1_ref, b1_ref, w2_ref, b2_ref,
             fdd_s):
    f32 = jnp.float32
    pc = pcol_ref[0]
    pr = prow_ref[:, 0, 0, :]
    ili = lax.broadcasted_iota(jnp.int32, (P, P), 1)
    isi = lax.broadcasted_iota(jnp.int32, (P, P), 0)
    d2 = ((pc[:, 0:1] - pr[0:1, :]) ** 2
          + (pc[:, 1:2] - pr[1:2, :]) ** 2
          + (pc[:, 2:3] - pr[2:3, :]) ** 2)
    cur = d2 + jnp.where(ili == isi, _BIG, 0.0)
    curp = _pack(cur, ili)

    l15 = lax.broadcasted_iota(jnp.int32, (1, 3 * KS), 1)
    gridf = (l15 % KS).astype(f32)
    s3 = lax.broadcasted_iota(jnp.int32, (3, 3 * KS), 0)
    rsel = (l15 // KS == s3).astype(f32)

    facc = jnp.zeros((P, 3 * FN), f32)
    idxs = []
    for _ in range(K):
        curp, idxi, mbf = _argmin_step(curp, ili)
        idxs.append(idxi)
        rel = _gather(mbf, pc) - pc
        u = (jnp.tanh(rel) + 1.0) * (0.5 * (KS - 1))
        urep = jnp.dot(u, rsel)
        basis = jnp.maximum(0.0, 1.0 - jnp.abs(urep - gridf))
        facc = facc + jnp.dot(basis, wm_ref[...])
    f3d = jax.nn.sigmoid(facc * (1.0 / K))

    w1 = w1_ref[...]
    gw = (jnp.dot(f3d, w1[:3 * FN, :])
          + jnp.dot(pc, w1[3 * FN:, :]))
    shift = jnp.dot(pc, w1[3 * FN:, :]) - b1_ref[...]
    w2 = w2_ref[...]
    b2 = b2_ref[...]
    fddm = jnp.full((P, 20), -jnp.inf, f32)
    for k in range(K):
        mbf = jnp.where(ili == idxs[k], 1.0, 0.0)
        h = jnp.maximum(_gather(mbf, gw) - shift, 0.0)
        h2 = jnp.maximum(jnp.dot(h, w2) + b2, 0.0)
        fddm = jnp.maximum(fddm, h2)
    fdd_s[pl.ds(g * P, P), :] = jax.nn.sigmoid(fddm)


def _fps(prow_ref, px2_s, py2_s, pz2_s):
    f32 = jnp.float32
    px = prow_ref[0]
    py = prow_ref[1]
    pz = prow_ref[2]
    lane = lax.broadcasted_iota(jnp.int32, (B, P), 1)
    lanerev = 1023 - lane
    lane2 = lax.broadcasted_iota(jnp.int32, (B, M), 1)
    pst = jnp.concatenate([px, py, pz], axis=0)
    cx0 = px[:, 0:1]
    cy0 = py[:, 0:1]
    cz0 = pz[:, 0:1]
    mind = (px - cx0) ** 2 + (py - cy0) ** 2 + (pz - cz0) ** 2
    p2x = jnp.where(lane2 == 0, cx0, 0.0)
    p2y = jnp.where(lane2 == 0, cy0, 0.0)
    p2z = jnp.where(lane2 == 0, cz0, 0.0)

    def body(i, st):
        mind, p2x, p2y, p2z = st
        mp = ((jax.lax.bitcast_convert_type(mind, jnp.int32)
               & -1024) | lanerev)
        mv = jnp.max(mp, axis=1, keepdims=True)
        ohf = jnp.where(mp == mv, 1.0, 0.0)
        oh3 = jnp.concatenate([ohf, ohf, ohf], axis=0)
        c3 = jnp.sum(pst * oh3, axis=1, keepdims=True)
        cx = c3[:B]
        cy = c3[B:2 * B]
        cz = c3[2 * B:]
        d = (px - cx) ** 2 + (py - cy) ** 2 + (pz - cz) ** 2
        mind = jnp.minimum(mind, d)
        sel = lane2 == i
        p2x = jnp.where(sel, cx, p2x)
        p2y = jnp.where(sel, cy, p2y)
        p2z = jnp.where(sel, cz, p2z)
        return (mind, p2x, p2y, p2z)

    _, p2x, p2y, p2z = lax.fori_loop(1, M, body, (mind, p2x, p2y, p2z),
                                     unroll=2)
    px2_s[...] = p2x
    py2_s[...] = p2y
    pz2_s[...] = p2z


def _stage_c(b, w3_ref, b3_ref, w4_ref, b4_ref, fdd_s, px2_s, py2_s, pz2_s,
             ys_s):
    f32 = jnp.float32
    bsel = lax.broadcasted_iota(jnp.int32, (B, 1), 0) == b
    prx = jnp.sum(jnp.where(bsel, px2_s[...], 0.0), axis=0, keepdims=True)
    pry = jnp.sum(jnp.where(bsel, py2_s[...], 0.0), axis=0, keepdims=True)
    prz = jnp.sum(jnp.where(bsel, pz2_s[...], 0.0), axis=0, keepdims=True)
    eyeb = (lax.broadcasted_iota(jnp.int32, (M, M), 1)
            == lax.broadcasted_iota(jnp.int32, (M, M), 0))
    pcx = jnp.sum(jnp.where(eyeb, prx, 0.0), axis=1, keepdims=True)
    pcy = jnp.sum(jnp.where(eyeb, pry, 0.0), axis=1, keepdims=True)
    pcz = jnp.sum(jnp.where(eyeb, prz, 0.0), axis=1, keepdims=True)
    d2 = (pcx - prx) ** 2 + (pcy - pry) ** 2 + (pcz - prz) ** 2
    ili = lax.broadcasted_iota(jnp.int32, (M, M), 1)
    curp = _pack(d2 + jnp.where(eyeb, _BIG, 0.0), ili)

    sub = fdd_s[pl.ds(b * M, M), :]
    macc = jnp.zeros((M, 20), f32)
    mmax = jnp.full((M, 20), -jnp.inf, f32)
    for _ in range(K):
        curp, _, mbf = _argmin_step(curp, ili)
        g = _gather(mbf, sub)
        macc = macc + g
        mmax = jnp.maximum(mmax, g)
    x2 = jnp.concatenate([sub, macc * (1.0 / K), mmax], axis=1)
    h = jnp.maximum(jnp.dot(x2, w3_ref[...]) + b3_ref[...], 0.0)
    h2 = jnp.maximum(jnp.dot(h, w4_ref[...]) + b4_ref[...], 0.0)
    f2 = jax.nn.sigmoid(h2)
    ysb = jnp.mean(f2, axis=0, keepdims=True)
    ys_s[...] = ys_s[...] + jnp.where(bsel, ysb, 0.0)


def _head(wn1_ref, bn1_ref, wn2_ref, bn2_ref, ys_s, out_ref):
    ys = ys_s[...]
    y1 = jnp.dot(ys, wn1_ref[...]) + bn1_ref[...]
    y1 = jnp.where(y1 > 0, y1, jnp.exp(jnp.minimum(y1, 0.0)) - 1.0)
    z = jnp.dot(y1, wn2_ref[...]) + bn2_ref[...]
    s = z - jnp.max(z, axis=1, keepdims=True)
    out_ref[...] = s - jnp.log(jnp.sum(jnp.exp(s), axis=1, keepdims=True))


def _body(pcol_ref, prow_all_ref, prow_g_ref, wm_ref, w1_ref, b1_ref,
          w2_ref, b2_ref, w3_ref, b3_ref, w4_ref, b4_ref,
          wn1_ref, bn1_ref, wn2_ref, bn2_ref,
          out_ref, fdd_s, px2_s, py2_s, pz2_s, ys_s):
    s = pl.program_id(0)

    @pl.when(s < NA)
    def _():
        _stage_a(s, pcol_ref, prow_g_ref, wm_ref, w1_ref, b1_ref,
                 w2_ref, b2_ref, fdd_s)

    @pl.when(s == NA)
    def _():
        ys_s[...] = jnp.zeros((B, 32), jnp.float32)
        _fps(prow_all_ref, px2_s, py2_s, pz2_s)

    @pl.when(s > NA)
    def _():
        _stage_c(s - (NA + 1), w3_ref, b3_ref, w4_ref, b4_ref,
                 fdd_s, px2_s, py2_s, pz2_s, ys_s)

    @pl.when(s == NA + B)
    def _():
        _head(wn1_ref, bn1_ref, wn2_ref, bn2_ref, ys_s, out_ref)


def kernel(pos, edge_index, batch, Wsp, W1, b1, W2, b2, W3, b3, W4, b4,
           Wn1, bn1, Wn2, bn2):
    del edge_index, batch
    f32 = jnp.float32
    pos = pos.astype(f32)
    posg = pos.reshape(B, P, 3)
    pg4 = posg[:NA]
    pall_row = jnp.transpose(posg, (2, 0, 1))
    pg4_row = pall_row[:, :NA, None, :]

    w_cjf = jnp.transpose(Wsp.astype(f32), (2, 1, 0))
    wmat = (w_cjf[:, :, :, None]
            * jnp.eye(3, dtype=f32)[:, None, None, :]).reshape(3 * KS, 3 * FN)

    nsteps = NA + 1 + B
    whole = lambda shape: pl.BlockSpec(shape, lambda s: (0,) * len(shape))
    out = pl.pallas_call(
        _body,
        grid=(nsteps,),
        in_specs=[
            pl.BlockSpec((1, P, 3), lambda s: (jnp.minimum(s, NA - 1), 0, 0)),
            whole((3, B, P)),
            pl.BlockSpec((3, 1, 1, P),
                         lambda s: (0, jnp.minimum(s, NA - 1), 0, 0)),
            whole((3 * KS, 3 * FN)),
            whole((3 * FN + 3, 20)),
            whole((1, 20)),
            whole((20, 20)),
            whole((1, 20)),
            whole((60, 64)),
            whole((1, 64)),
            whole((64, 32)),
            whole((1, 32)),
            whole((32, 256)),
            whole((1, 256)),
            whole((256, 40)),
            whole((1, 40)),
        ],
        out_specs=pl.BlockSpec((B, 40), lambda s: (0, 0)),
        out_shape=jax.ShapeDtypeStruct((B, 40), f32),
        scratch_shapes=[
            pltpu.VMEM((NA * P, 20), f32),
            pltpu.VMEM((B, M), f32),
            pltpu.VMEM((B, M), f32),
            pltpu.VMEM((B, M), f32),
            pltpu.VMEM((B, 32), f32),
        ],
    )(pg4, pall_row, pg4_row, wmat, W1, b1.reshape(1, 20), W2,
      b2.reshape(1, 20), W3, b3.reshape(1, 64), W4, b4.reshape(1, 32),
      Wn1, bn1.reshape(1, 256), Wn2, bn2.reshape(1, 40))
    return out
